# Initial kernel scaffold; baseline (speedup 1.0000x reference)
import jax
import jax.numpy as jnp
from jax.experimental import pallas as pl


def kernel(x, edge_index, W_in, b_in, W_out, b_out):
    raise NotImplementedError("write your pallas kernel here")


# speedup vs baseline: 8.4673x; 8.4673x over previous
import functools

import jax
import jax.numpy as jnp
from jax import lax
from jax.experimental import pallas as pl
from jax.experimental.pallas import tpu as pltpu
from jax.experimental.pallas import tpu_sc as plsc

N_ = 10000
E_ = 320000
F_ = 128
H_ = 128
C_ = 64
L_ = 10
ALPHA_ = 0.9
BETA_ = 1.0 - ALPHA_

NSC_ = 2
NT_ = 16
CH_ = C_ // NSC_
RPT_ = 640
NPAD_ = NT_ * RPT_
KE_ = 128
EPT_ = -(-E_ // NT_)
NCH_ = -(-EPT_ // KE_)
EPAD_ = NT_ * NCH_ * KE_


def _mlp_body(x_ref, wi_ref, bi_ref, wo_ref, bo_ref, o_ref):
    h = lax.dot_general(x_ref[...], wi_ref[...], (((1,), (1,)), ((), ())),
                        preferred_element_type=jnp.float32)
    h = jnp.maximum(h + bi_ref[...], 0.0)
    o = lax.dot_general(h, wo_ref[...], (((1,), (1,)), ((), ())),
                        preferred_element_type=jnp.float32)
    o_ref[...] = o + bo_ref[...]


def _mlp(x_pad, W_in, b_in, W_out, b_out):
    grid = NPAD_ // RPT_
    return pl.pallas_call(
        _mlp_body,
        grid=(grid,),
        in_specs=[
            pl.BlockSpec((RPT_, F_), lambda i: (i, 0)),
            pl.BlockSpec((H_, F_), lambda i: (0, 0)),
            pl.BlockSpec((1, H_), lambda i: (0, 0)),
            pl.BlockSpec((C_, H_), lambda i: (0, 0)),
            pl.BlockSpec((1, C_), lambda i: (0, 0)),
        ],
        out_specs=pl.BlockSpec((RPT_, C_), lambda i: (i, 0)),
        out_shape=jax.ShapeDtypeStruct((NPAD_, C_), jnp.float32),
    )(x_pad, W_in, b_in.reshape(1, H_), W_out, b_out.reshape(1, C_))


def _fill(ref, rows, val):
    v16 = jnp.full((16,), val, jnp.float32)

    def fb(r, carry):
        ref[r, pl.ds(0, 16)] = v16
        ref[r, pl.ds(16, 16)] = v16
        return carry

    lax.fori_loop(0, rows, fb, 0)


def _sc_body(x0buf, srcg, dstg, hbuf,
             src_scr, dst_scr, gbuf, cbuf, accbuf, normbuf, bx0buf, acc):
    c = lax.axis_index("c")
    s = lax.axis_index("s")
    row0 = c * NPAD_
    base = s * RPT_

    pltpu.sync_copy(srcg.at[c, s], src_scr)
    pltpu.sync_copy(dstg.at[s], dst_scr)

    pltpu.sync_copy(x0buf.at[pl.ds(row0 + base, RPT_)], bx0buf)
    pltpu.sync_copy(bx0buf, hbuf.at[pl.ds(row0 + base, RPT_)])

    _fill(cbuf, KE_, 0.0)
    for k in range(RPT_ // KE_):
        pltpu.sync_copy(cbuf, acc.at[pl.ds(base + k * KE_, KE_)])
    plsc.subcore_barrier()

    _fill(cbuf, KE_, 1.0)

    def deg_body(j, carry):
        pltpu.sync_copy(cbuf, acc.at[dst_scr.at[j]], add=True)
        return carry

    lax.fori_loop(0, NCH_, deg_body, 0)
    plsc.subcore_barrier()

    pltpu.sync_copy(acc.at[pl.ds(base, RPT_)], accbuf)

    def norm_body(v, carry):
        for col in (0, 16):
            d = accbuf[v, pl.ds(col, 16)]
            normbuf[v, pl.ds(col, 16)] = ALPHA_ / jnp.maximum(d, 1.0)
            b = bx0buf[v, pl.ds(col, 16)]
            bx0buf[v, pl.ds(col, 16)] = b * BETA_
        return carry

    lax.fori_loop(0, RPT_, norm_body, 0)

    _fill(cbuf, KE_, 0.0)
    for k in range(RPT_ // KE_):
        pltpu.sync_copy(cbuf, acc.at[pl.ds(base + k * KE_, KE_)])
    plsc.subcore_barrier()

    def round_body(r, carry):
        def edge_body(j, carry2):
            pltpu.sync_copy(hbuf.at[src_scr.at[j]], gbuf)
            pltpu.sync_copy(gbuf, acc.at[dst_scr.at[j]], add=True)
            return carry2

        lax.fori_loop(0, NCH_, edge_body, 0)
        plsc.subcore_barrier()

        pltpu.sync_copy(acc.at[pl.ds(base, RPT_)], accbuf)
        for k in range(RPT_ // KE_):
            pltpu.sync_copy(cbuf, acc.at[pl.ds(base + k * KE_, KE_)])

        def hb(v, carry2):
            for col in (0, 16):
                a = accbuf[v, pl.ds(col, 16)]
                nr = normbuf[v, pl.ds(col, 16)]
                b = bx0buf[v, pl.ds(col, 16)]
                accbuf[v, pl.ds(col, 16)] = a * nr + b
            return carry2

        lax.fori_loop(0, RPT_, hb, 0)
        pltpu.sync_copy(accbuf, hbuf.at[pl.ds(row0 + base, RPT_)])
        plsc.subcore_barrier()
        return carry

    lax.fori_loop(0, L_, round_body, 0)


def _propagate(x0buf, srcg, dstg):
    mesh = plsc.VectorSubcoreMesh(core_axis_name="c", subcore_axis_name="s")
    return pl.kernel(
        _sc_body,
        out_type=jax.ShapeDtypeStruct((NSC_ * NPAD_, CH_), jnp.float32),
        mesh=mesh,
        compiler_params=pltpu.CompilerParams(use_tc_tiling_on_sc=False),
        scratch_types=[
            pltpu.VMEM((NCH_, KE_), jnp.int32),
            pltpu.VMEM((NCH_, KE_), jnp.int32),
            pltpu.VMEM((KE_, CH_), jnp.float32),
            pltpu.VMEM((KE_, CH_), jnp.float32),
            pltpu.VMEM((RPT_, CH_), jnp.float32),
            pltpu.VMEM((RPT_, CH_), jnp.float32),
            pltpu.VMEM((RPT_, CH_), jnp.float32),
            pltpu.VMEM_SHARED((NPAD_, CH_), jnp.float32),
        ],
    )(x0buf, srcg, dstg)


def kernel(x, edge_index, W_in, b_in, W_out, b_out):
    x_pad = jnp.concatenate(
        [x, jnp.zeros((NPAD_ - N_, F_), jnp.float32)], axis=0)
    x0 = _mlp(x_pad, W_in, b_in, W_out, b_out)
    x0buf = x0.reshape(NPAD_, NSC_, CH_).transpose(1, 0, 2).reshape(
        NSC_ * NPAD_, CH_)

    src = edge_index[0]
    dst = edge_index[1]
    pad = EPAD_ - E_
    src_p = jnp.concatenate([src, jnp.zeros((pad,), jnp.int32)])
    dst_p = jnp.concatenate([dst, jnp.full((pad,), N_, jnp.int32)])
    srcg = jnp.stack([src_p, src_p + NPAD_]).reshape(NSC_, NT_, NCH_, KE_)
    dstg = dst_p.reshape(NT_, NCH_, KE_)

    hbuf = _propagate(x0buf, srcg, dstg)
    h = hbuf.reshape(NSC_, NPAD_, CH_).transpose(1, 0, 2).reshape(NPAD_, C_)
    return h[:N_]
